# Initial kernel scaffold; baseline (speedup 1.0000x reference)
import jax
import jax.numpy as jnp
from jax.experimental import pallas as pl


def kernel(atom_features, bond_features, adjacency_matrix, batch_indices, atom_emb, bond_emb, msg_W, msg_b, gru_Wih, gru_Whh, gru_bih, gru_bhh, pool_W, pool_b):
    raise NotImplementedError("write your pallas kernel here")


# speedup vs baseline: 1.3217x; 1.3217x over previous
import jax
import jax.numpy as jnp
from jax.experimental import pallas as pl
from jax.experimental.pallas import tpu as pltpu

N = 4096
NB = 8192
E = 32
H = 256
DEPTH = 10
BATCH = 64
NUM_ATOM_TYPES = 200
NUM_BOND_TYPES = 10

_ROWS = 512
_NBLK = N // _ROWS


def _prep_body(adj_ref, mask_ref, scale_ref, flag_ref):
    a = adj_ref[...]
    m = a != 0
    mask_ref[...] = m.astype(jnp.bfloat16)
    deg = jnp.sum(m.astype(jnp.float32), axis=1, keepdims=True)
    has = deg > 0.0
    safe = jnp.where(has, deg, 1.0)
    scale_ref[...] = jnp.where(has, 1.0 / safe, 0.0)
    flag_ref[...] = has.astype(jnp.float32)


def _mpn_body(mask_ref, scale_ref, flag_ref, af_ref, bi_ref, atom_emb_ref,
              msgWT_ref, mb_ref, WihT_ref, WhhT_ref, bih_ref, bhh_ref,
              poolWT_ref, pb_ref,
              h_out_ref, graph_ref,
              buf_ref, h2_ref, C_ref, row_ref):
    l = pl.program_id(0)
    b = pl.program_id(1)
    f32 = jnp.float32

    @pl.when(l == 0)
    def _init():
        af = af_ref[...]
        iota = jax.lax.broadcasted_iota(jnp.int32, (_ROWS, NUM_ATOM_TYPES), 1)
        oh = (af == iota).astype(f32)
        h0 = jnp.dot(oh, atom_emb_ref[...], preferred_element_type=f32)
        buf_ref[0, pl.ds(b * _ROWS, _ROWS), :] = h0

    @pl.when(jnp.logical_and(l > 0, b == 0))
    def _layer_setup():
        hprev = buf_ref[(l - 1) % 2, :, :]
        hi = hprev.astype(jnp.bfloat16)
        lo = (hprev - hi.astype(f32)).astype(jnp.bfloat16)
        h2_ref[...] = jnp.concatenate([hi, lo], axis=1)
        Wih = WihT_ref[0]
        A = jnp.dot(msgWT_ref[0, :E, :], Wih, preferred_element_type=f32)
        B = jnp.dot(msgWT_ref[0, E:, :], Wih, preferred_element_type=f32)
        Whh = WhhT_ref[0]
        top = jnp.concatenate([A, Whh], axis=1)
        bot = jnp.concatenate([B, jnp.zeros((E, 3 * E), f32)], axis=1)
        C_ref[...] = jnp.concatenate([top, bot], axis=0)
        row_ref[...] = jnp.dot(mb_ref[0], Wih, preferred_element_type=f32)

    @pl.when(l > 0)
    def _layer():
        acc = jnp.dot(mask_ref[...], h2_ref[...], preferred_element_type=f32)
        agg = acc[:, :E] + acc[:, E:]
        aggs = agg * scale_ref[...]
        hb = buf_ref[(l - 1) % 2, pl.ds(b * _ROWS, _ROWS), :]
        X = jnp.concatenate([hb, aggs], axis=1)
        G = jnp.dot(X, C_ref[...], preferred_element_type=f32)
        gi = flag_ref[...] * (G[:, :3 * E] + row_ref[...]) + bih_ref[0]
        gh = G[:, 3 * E:] + bhh_ref[0]
        r = jax.nn.sigmoid(gi[:, :E] + gh[:, :E])
        z = jax.nn.sigmoid(gi[:, E:2 * E] + gh[:, E:2 * E])
        nn = jnp.tanh(gi[:, 2 * E:] + r * gh[:, 2 * E:])
        hn = (1.0 - z) * nn + z * hb
        buf_ref[l % 2, pl.ds(b * _ROWS, _ROWS), :] = hn

        @pl.when(l == DEPTH)
        def _emit():
            h_out_ref[...] = hn

    @pl.when(jnp.logical_and(l == DEPTH, b == _NBLK - 1))
    def _pool():
        hfin = buf_ref[DEPTH % 2, :, :]
        bi = bi_ref[...]
        iota = jax.lax.broadcasted_iota(jnp.int32, (BATCH, N), 0)
        P = (iota == bi).astype(f32)
        counts = jnp.sum(P, axis=1, keepdims=True)
        sums = jnp.dot(P, hfin, preferred_element_type=f32)
        safe_c = jnp.where(counts > 0.0, counts, 1.0)
        means = jnp.where(counts > 0.0, sums / safe_c, 0.0)
        graph_ref[...] = (jnp.dot(means, poolWT_ref[...],
                                  preferred_element_type=f32) + pb_ref[...])


def _bond_body(bf_ref, bond_emb_ref, out_ref):
    bfi = bf_ref[...]
    iota = jax.lax.broadcasted_iota(jnp.int32, (NB, NUM_BOND_TYPES), 1)
    oh = (bfi == iota).astype(jnp.float32)
    out_ref[...] = jnp.dot(oh, bond_emb_ref[...],
                           preferred_element_type=jnp.float32)


def kernel(atom_features, bond_features, adjacency_matrix, batch_indices,
           atom_emb, bond_emb, msg_W, msg_b,
           gru_Wih, gru_Whh, gru_bih, gru_bhh, pool_W, pool_b):
    mask, scale, flag = pl.pallas_call(
        _prep_body,
        grid=(_NBLK,),
        in_specs=[pl.BlockSpec((_ROWS, N), lambda i: (i, 0))],
        out_specs=[
            pl.BlockSpec((_ROWS, N), lambda i: (i, 0)),
            pl.BlockSpec((_ROWS, 1), lambda i: (i, 0)),
            pl.BlockSpec((_ROWS, 1), lambda i: (i, 0)),
        ],
        out_shape=[
            jax.ShapeDtypeStruct((N, N), jnp.bfloat16),
            jax.ShapeDtypeStruct((N, 1), jnp.float32),
            jax.ShapeDtypeStruct((N, 1), jnp.float32),
        ],
    )(adjacency_matrix)

    af = atom_features.reshape(N, 1).astype(jnp.int32)
    bf = bond_features.reshape(NB, 1).astype(jnp.int32)
    bi = batch_indices.reshape(1, N).astype(jnp.int32)
    msgWT = msg_W.transpose(0, 2, 1)
    WihT = gru_Wih.transpose(0, 2, 1)
    WhhT = gru_Whh.transpose(0, 2, 1)
    mb3 = msg_b.reshape(DEPTH, 1, H)
    bih3 = gru_bih.reshape(DEPTH, 1, 3 * E)
    bhh3 = gru_bhh.reshape(DEPTH, 1, 3 * E)
    poolWT = pool_W.T
    pb = pool_b.reshape(1, H)

    def _wmap(nd):
        def im(l, b):
            lw = jnp.maximum(l - 1, 0)
            return (lw,) + (0,) * (nd - 1)
        return im

    h_out, graph = pl.pallas_call(
        _mpn_body,
        grid=(DEPTH + 1, _NBLK),
        in_specs=[
            pl.BlockSpec((_ROWS, N), lambda l, b: (b, 0)),
            pl.BlockSpec((_ROWS, 1), lambda l, b: (b, 0)),
            pl.BlockSpec((_ROWS, 1), lambda l, b: (b, 0)),
            pl.BlockSpec((_ROWS, 1), lambda l, b: (b, 0)),
            pl.BlockSpec((1, N), lambda l, b: (0, 0)),
            pl.BlockSpec((NUM_ATOM_TYPES, E), lambda l, b: (0, 0)),
            pl.BlockSpec((1, 2 * E, H), _wmap(3)),
            pl.BlockSpec((1, 1, H), _wmap(3)),
            pl.BlockSpec((1, H, 3 * E), _wmap(3)),
            pl.BlockSpec((1, E, 3 * E), _wmap(3)),
            pl.BlockSpec((1, 1, 3 * E), _wmap(3)),
            pl.BlockSpec((1, 1, 3 * E), _wmap(3)),
            pl.BlockSpec((E, H), lambda l, b: (0, 0)),
            pl.BlockSpec((1, H), lambda l, b: (0, 0)),
        ],
        out_specs=[
            pl.BlockSpec((_ROWS, E), lambda l, b: (b, 0)),
            pl.BlockSpec((BATCH, H), lambda l, b: (0, 0)),
        ],
        out_shape=[
            jax.ShapeDtypeStruct((N, E), jnp.float32),
            jax.ShapeDtypeStruct((BATCH, H), jnp.float32),
        ],
        scratch_shapes=[
            pltpu.VMEM((2, N, E), jnp.float32),
            pltpu.VMEM((N, 2 * E), jnp.bfloat16),
            pltpu.VMEM((2 * E, 6 * E), jnp.float32),
            pltpu.VMEM((1, 3 * E), jnp.float32),
        ],
    )(mask, scale, flag, af, bi, atom_emb, msgWT, mb3, WihT, WhhT,
      bih3, bhh3, poolWT, pb)

    bond_out = pl.pallas_call(
        _bond_body,
        out_shape=jax.ShapeDtypeStruct((NB, E), jnp.float32),
    )(bf, bond_emb)

    return (h_out, bond_out, graph)


# speedup vs baseline: 1.4821x; 1.1213x over previous
import jax
import jax.numpy as jnp
from jax.experimental import pallas as pl
from jax.experimental.pallas import tpu as pltpu

N = 4096
NB = 8192
E = 32
H = 256
DEPTH = 10
BATCH = 64
NUM_ATOM_TYPES = 200
NUM_BOND_TYPES = 10

_ROWS = 512
_NBLK = N // _ROWS


def _prep_body(adj_ref, mask_ref, scale_ref, flag_ref):
    a = adj_ref[...]
    m = a != 0
    mask_ref[...] = m.astype(jnp.bfloat16)
    deg = jnp.sum(m.astype(jnp.float32), axis=1, keepdims=True)
    has = deg > 0.0
    safe = jnp.where(has, deg, 1.0)
    scale_ref[...] = jnp.where(has, 1.0 / safe, 0.0)
    flag_ref[...] = has.astype(jnp.float32)


def _mpn_body(mask_ref, scale_ref, flag_ref, af_ref, bi_ref, atom_embT_ref,
              msgW_ref, mbc_ref, Wih_ref, Whh_ref, bihc_ref, bhhc_ref,
              poolW_ref, pbc_ref,
              hT_out_ref, graphT_ref,
              bufT_ref, h2_ref, agg_ref):
    l = pl.program_id(0)
    b = pl.program_id(1)
    f32 = jnp.float32
    bf16 = jnp.bfloat16

    @pl.when(jnp.logical_and(l == 0, b == 0))
    def _init():
        af = af_ref[...]
        iota = jax.lax.broadcasted_iota(jnp.int32, (NUM_ATOM_TYPES, N), 0)
        ohT = (iota == af).astype(f32)
        h0T = jnp.dot(atom_embT_ref[...], ohT, preferred_element_type=f32)
        bufT_ref[0] = h0T
        hi = h0T.astype(bf16)
        lo = (h0T - hi.astype(f32)).astype(bf16)
        h2_ref[...] = jnp.transpose(jnp.concatenate([hi, lo], axis=0))

    @pl.when(l > 0)
    def _layer():
        acc = jnp.dot(mask_ref[...], h2_ref[...], preferred_element_type=f32)
        agg_ref[pl.ds(b * _ROWS, _ROWS), :] = acc

        @pl.when(b == _NBLK - 1)
        def _gates():
            Wih = Wih_ref[0]
            W = msgW_ref[0]
            A_T = jnp.dot(Wih, W[:, :E], preferred_element_type=f32)
            B_T = jnp.dot(Wih, W[:, E:], preferred_element_type=f32)
            top = jnp.concatenate([A_T, B_T], axis=1)
            bot = jnp.concatenate([Whh_ref[0], jnp.zeros((3 * E, E), f32)],
                                  axis=1)
            CT = jnp.concatenate([top, bot], axis=0)
            CThi = CT.astype(bf16)
            CTlo = (CT - CThi.astype(f32)).astype(bf16)
            CT2 = jnp.concatenate([CThi, CTlo], axis=0)
            rowT = jnp.dot(Wih, mbc_ref[0], preferred_element_type=f32)

            aggT = jnp.transpose(agg_ref[...])
            aggsT = (aggT[:E] + aggT[E:]) * scale_ref[...]
            hT = bufT_ref[(l - 1) % 2]
            XT = jnp.concatenate([hT, aggsT], axis=0)
            XThi = XT.astype(bf16)
            XTlo = (XT - XThi.astype(f32)).astype(bf16)
            G1 = jnp.dot(CT2, XThi, preferred_element_type=f32)
            G2 = jnp.dot(CThi, XTlo, preferred_element_type=f32)
            GT = G1[:6 * E] + G1[6 * E:] + G2
            giT = flag_ref[...] * (GT[:3 * E] + rowT) + bihc_ref[0]
            ghT = GT[3 * E:] + bhhc_ref[0]
            rT = jax.nn.sigmoid(giT[:E] + ghT[:E])
            zT = jax.nn.sigmoid(giT[E:2 * E] + ghT[E:2 * E])
            nT = jnp.tanh(giT[2 * E:] + rT * ghT[2 * E:])
            hnT = (1.0 - zT) * nT + zT * hT
            bufT_ref[l % 2] = hnT
            hi = hnT.astype(bf16)
            lo = (hnT - hi.astype(f32)).astype(bf16)
            h2_ref[...] = jnp.transpose(jnp.concatenate([hi, lo], axis=0))

            @pl.when(l == DEPTH)
            def _emit():
                hT_out_ref[...] = hnT
                bi = bi_ref[...]
                iota = jax.lax.broadcasted_iota(jnp.int32, (N, BATCH), 1)
                PT = (iota == bi).astype(f32)
                countsT = jnp.sum(PT, axis=0, keepdims=True)
                sumsT = jnp.dot(hnT, PT, preferred_element_type=f32)
                inv = jnp.where(countsT > 0.0,
                                1.0 / jnp.where(countsT > 0.0, countsT, 1.0),
                                0.0)
                meansT = sumsT * inv
                graphT_ref[...] = (jnp.dot(poolW_ref[...], meansT,
                                           preferred_element_type=f32)
                                   + pbc_ref[...])


def _bond_body(bf_ref, bond_emb_ref, out_ref):
    bfi = bf_ref[...]
    iota = jax.lax.broadcasted_iota(jnp.int32, (NB, NUM_BOND_TYPES), 1)
    oh = (bfi == iota).astype(jnp.float32)
    out_ref[...] = jnp.dot(oh, bond_emb_ref[...],
                           preferred_element_type=jnp.float32)


def kernel(atom_features, bond_features, adjacency_matrix, batch_indices,
           atom_emb, bond_emb, msg_W, msg_b,
           gru_Wih, gru_Whh, gru_bih, gru_bhh, pool_W, pool_b):
    mask, scale, flag = pl.pallas_call(
        _prep_body,
        grid=(_NBLK,),
        in_specs=[pl.BlockSpec((_ROWS, N), lambda i: (i, 0))],
        out_specs=[
            pl.BlockSpec((_ROWS, N), lambda i: (i, 0)),
            pl.BlockSpec((_ROWS, 1), lambda i: (i, 0)),
            pl.BlockSpec((_ROWS, 1), lambda i: (i, 0)),
        ],
        out_shape=[
            jax.ShapeDtypeStruct((N, N), jnp.bfloat16),
            jax.ShapeDtypeStruct((N, 1), jnp.float32),
            jax.ShapeDtypeStruct((N, 1), jnp.float32),
        ],
    )(adjacency_matrix)

    scaleR = scale.reshape(1, N)
    flagR = flag.reshape(1, N)
    af = atom_features.reshape(1, N).astype(jnp.int32)
    bf = bond_features.reshape(NB, 1).astype(jnp.int32)
    bi = batch_indices.reshape(N, 1).astype(jnp.int32)
    atom_embT = atom_emb.T
    msgWn = msg_W
    mbc = msg_b.reshape(DEPTH, H, 1)
    bihc = gru_bih.reshape(DEPTH, 3 * E, 1)
    bhhc = gru_bhh.reshape(DEPTH, 3 * E, 1)
    pbc = pool_b.reshape(H, 1)

    def _wmap(nd):
        def im(l, b):
            lw = jnp.maximum(l - 1, 0)
            return (lw,) + (0,) * (nd - 1)
        return im

    hT_out, graphT = pl.pallas_call(
        _mpn_body,
        grid=(DEPTH + 1, _NBLK),
        in_specs=[
            pl.BlockSpec((_ROWS, N),
                         lambda l, b: (jnp.where(l == 0, 0, b), 0)),
            pl.BlockSpec((1, N), lambda l, b: (0, 0)),
            pl.BlockSpec((1, N), lambda l, b: (0, 0)),
            pl.BlockSpec((1, N), lambda l, b: (0, 0)),
            pl.BlockSpec((N, 1), lambda l, b: (0, 0)),
            pl.BlockSpec((E, NUM_ATOM_TYPES), lambda l, b: (0, 0)),
            pl.BlockSpec((1, H, 2 * E), _wmap(3)),
            pl.BlockSpec((1, H, 1), _wmap(3)),
            pl.BlockSpec((1, 3 * E, H), _wmap(3)),
            pl.BlockSpec((1, 3 * E, E), _wmap(3)),
            pl.BlockSpec((1, 3 * E, 1), _wmap(3)),
            pl.BlockSpec((1, 3 * E, 1), _wmap(3)),
            pl.BlockSpec((H, E), lambda l, b: (0, 0)),
            pl.BlockSpec((H, 1), lambda l, b: (0, 0)),
        ],
        out_specs=[
            pl.BlockSpec((E, N), lambda l, b: (0, 0)),
            pl.BlockSpec((H, BATCH), lambda l, b: (0, 0)),
        ],
        out_shape=[
            jax.ShapeDtypeStruct((E, N), jnp.float32),
            jax.ShapeDtypeStruct((H, BATCH), jnp.float32),
        ],
        scratch_shapes=[
            pltpu.VMEM((2, E, N), jnp.float32),
            pltpu.VMEM((N, 2 * E), jnp.bfloat16),
            pltpu.VMEM((N, 2 * E), jnp.float32),
        ],
    )(mask, scaleR, flagR, af, bi, atom_embT, msgWn, mbc, gru_Wih, gru_Whh,
      bihc, bhhc, pool_W, pbc)

    bond_out = pl.pallas_call(
        _bond_body,
        out_shape=jax.ShapeDtypeStruct((NB, E), jnp.float32),
    )(bf, bond_emb)

    return (hT_out.T, bond_out, graphT.T)


# speedup vs baseline: 2.2234x; 1.5002x over previous
import jax
import jax.numpy as jnp
from jax.experimental import pallas as pl
from jax.experimental.pallas import tpu as pltpu

N = 4096
NB = 8192
E = 32
H = 256
DEPTH = 10
BATCH = 64
NUM_ATOM_TYPES = 200
NUM_BOND_TYPES = 10

_ROWS = 512
_NBLK = N // _ROWS


def _prep_body(adj_ref, mask_ref, scale_ref, flag_ref):
    a = adj_ref[...]
    m = a != 0
    mask_ref[...] = m.astype(jnp.float8_e4m3fn)
    deg = jnp.sum(m.astype(jnp.float32), axis=1, keepdims=True)
    has = deg > 0.0
    safe = jnp.where(has, deg, 1.0)
    scale_ref[...] = jnp.where(has, 1.0 / safe, 0.0)
    flag_ref[...] = has.astype(jnp.float32)


def _store_h2(h2_ref, xT):
    f32 = jnp.float32
    f8 = jnp.float8_e4m3fn
    hN = jnp.transpose(xT)
    t0 = hN.astype(f8)
    r0 = hN - t0.astype(f32)
    t1 = (r0 * 256.0).astype(f8)
    r1 = r0 - t1.astype(f32) * (1.0 / 256.0)
    t2 = (r1 * 65536.0).astype(f8)
    h2_ref[...] = jnp.concatenate([t0, t1, t2], axis=1)


def _mpn_body(mask_ref, scale_ref, flag_ref, af_ref, bi_ref, atom_embT_ref,
              msgW_ref, mbc_ref, Wih_ref, Whh_ref, bihc_ref, bhhc_ref,
              poolW_ref, pbc_ref,
              hT_out_ref, graphT_ref,
              bufT_ref, h2_ref, agg_ref):
    l = pl.program_id(0)
    b = pl.program_id(1)
    f32 = jnp.float32
    bf16 = jnp.bfloat16

    @pl.when(jnp.logical_and(l == 0, b == 0))
    def _init():
        af = af_ref[...]
        iota = jax.lax.broadcasted_iota(jnp.int32, (NUM_ATOM_TYPES, N), 0)
        ohT = (iota == af).astype(f32)
        h0T = jnp.dot(atom_embT_ref[...], ohT, preferred_element_type=f32)
        bufT_ref[0] = h0T
        _store_h2(h2_ref, h0T)

    @pl.when(l > 0)
    def _layer():
        acc = jnp.dot(mask_ref[...], h2_ref[...], preferred_element_type=f32)
        agg_ref[pl.ds(b * _ROWS, _ROWS), :] = acc

        @pl.when(b == _NBLK - 1)
        def _gates():
            Wih = Wih_ref[0]
            W = msgW_ref[0]
            A_T = jnp.dot(Wih, W[:, :E], preferred_element_type=f32)
            B_T = jnp.dot(Wih, W[:, E:], preferred_element_type=f32)
            top = jnp.concatenate([A_T, B_T], axis=1)
            bot = jnp.concatenate([Whh_ref[0], jnp.zeros((3 * E, E), f32)],
                                  axis=1)
            CT = jnp.concatenate([top, bot], axis=0)
            CThi = CT.astype(bf16)
            CTlo = (CT - CThi.astype(f32)).astype(bf16)
            CT2 = jnp.concatenate([CThi, CTlo], axis=0)
            rowT = jnp.dot(Wih, mbc_ref[0], preferred_element_type=f32)

            aggT = jnp.transpose(agg_ref[...])
            agg1 = (aggT[:E] + aggT[E:2 * E] * (1.0 / 256.0)
                    + aggT[2 * E:] * (1.0 / 65536.0))
            aggsT = agg1 * scale_ref[...]
            hT = bufT_ref[(l - 1) % 2]
            XT = jnp.concatenate([hT, aggsT], axis=0)
            XThi = XT.astype(bf16)
            XTlo = (XT - XThi.astype(f32)).astype(bf16)
            G1 = jnp.dot(CT2, XThi, preferred_element_type=f32)
            G2 = jnp.dot(CThi, XTlo, preferred_element_type=f32)
            GT = G1[:6 * E] + G1[6 * E:] + G2
            giT = flag_ref[...] * (GT[:3 * E] + rowT) + bihc_ref[0]
            ghT = GT[3 * E:] + bhhc_ref[0]
            rT = jax.nn.sigmoid(giT[:E] + ghT[:E])
            zT = jax.nn.sigmoid(giT[E:2 * E] + ghT[E:2 * E])
            nT = jnp.tanh(giT[2 * E:] + rT * ghT[2 * E:])
            hnT = (1.0 - zT) * nT + zT * hT
            bufT_ref[l % 2] = hnT
            _store_h2(h2_ref, hnT)

            @pl.when(l == DEPTH)
            def _emit():
                hT_out_ref[...] = hnT
                bi = bi_ref[...]
                iota = jax.lax.broadcasted_iota(jnp.int32, (N, BATCH), 1)
                PT = (iota == bi).astype(f32)
                countsT = jnp.sum(PT, axis=0, keepdims=True)
                sumsT = jnp.dot(hnT, PT, preferred_element_type=f32)
                inv = jnp.where(countsT > 0.0,
                                1.0 / jnp.where(countsT > 0.0, countsT, 1.0),
                                0.0)
                meansT = sumsT * inv
                graphT_ref[...] = (jnp.dot(poolW_ref[...], meansT,
                                           preferred_element_type=f32)
                                   + pbc_ref[...])


def _bond_body(bf_ref, bond_emb_ref, out_ref):
    bfi = bf_ref[...]
    iota = jax.lax.broadcasted_iota(jnp.int32, (NB, NUM_BOND_TYPES), 1)
    oh = (bfi == iota).astype(jnp.float32)
    out_ref[...] = jnp.dot(oh, bond_emb_ref[...],
                           preferred_element_type=jnp.float32)


def kernel(atom_features, bond_features, adjacency_matrix, batch_indices,
           atom_emb, bond_emb, msg_W, msg_b,
           gru_Wih, gru_Whh, gru_bih, gru_bhh, pool_W, pool_b):
    mask, scale, flag = pl.pallas_call(
        _prep_body,
        grid=(_NBLK,),
        in_specs=[pl.BlockSpec((_ROWS, N), lambda i: (i, 0))],
        out_specs=[
            pl.BlockSpec((_ROWS, N), lambda i: (i, 0)),
            pl.BlockSpec((_ROWS, 1), lambda i: (i, 0)),
            pl.BlockSpec((_ROWS, 1), lambda i: (i, 0)),
        ],
        out_shape=[
            jax.ShapeDtypeStruct((N, N), jnp.float8_e4m3fn),
            jax.ShapeDtypeStruct((N, 1), jnp.float32),
            jax.ShapeDtypeStruct((N, 1), jnp.float32),
        ],
    )(adjacency_matrix)

    scaleR = scale.reshape(1, N)
    flagR = flag.reshape(1, N)
    af = atom_features.reshape(1, N).astype(jnp.int32)
    bf = bond_features.reshape(NB, 1).astype(jnp.int32)
    bi = batch_indices.reshape(N, 1).astype(jnp.int32)
    atom_embT = atom_emb.T
    msgWn = msg_W
    mbc = msg_b.reshape(DEPTH, H, 1)
    bihc = gru_bih.reshape(DEPTH, 3 * E, 1)
    bhhc = gru_bhh.reshape(DEPTH, 3 * E, 1)
    pbc = pool_b.reshape(H, 1)

    def _wmap(nd):
        def im(l, b):
            lw = jnp.maximum(l - 1, 0)
            return (lw,) + (0,) * (nd - 1)
        return im

    hT_out, graphT = pl.pallas_call(
        _mpn_body,
        grid=(DEPTH + 1, _NBLK),
        in_specs=[
            pl.BlockSpec((_ROWS, N),
                         lambda l, b: (0, 0)),
            pl.BlockSpec((1, N), lambda l, b: (0, 0)),
            pl.BlockSpec((1, N), lambda l, b: (0, 0)),
            pl.BlockSpec((1, N), lambda l, b: (0, 0)),
            pl.BlockSpec((N, 1), lambda l, b: (0, 0)),
            pl.BlockSpec((E, NUM_ATOM_TYPES), lambda l, b: (0, 0)),
            pl.BlockSpec((1, H, 2 * E), _wmap(3)),
            pl.BlockSpec((1, H, 1), _wmap(3)),
            pl.BlockSpec((1, 3 * E, H), _wmap(3)),
            pl.BlockSpec((1, 3 * E, E), _wmap(3)),
            pl.BlockSpec((1, 3 * E, 1), _wmap(3)),
            pl.BlockSpec((1, 3 * E, 1), _wmap(3)),
            pl.BlockSpec((H, E), lambda l, b: (0, 0)),
            pl.BlockSpec((H, 1), lambda l, b: (0, 0)),
        ],
        out_specs=[
            pl.BlockSpec((E, N), lambda l, b: (0, 0)),
            pl.BlockSpec((H, BATCH), lambda l, b: (0, 0)),
        ],
        out_shape=[
            jax.ShapeDtypeStruct((E, N), jnp.float32),
            jax.ShapeDtypeStruct((H, BATCH), jnp.float32),
        ],
        scratch_shapes=[
            pltpu.VMEM((2, E, N), jnp.float32),
            pltpu.VMEM((N, 3 * E), jnp.float8_e4m3fn),
            pltpu.VMEM((N, 3 * E), jnp.float32),
        ],
    )(mask, scaleR, flagR, af, bi, atom_embT, msgWn, mbc, gru_Wih, gru_Whh,
      bihc, bhhc, pool_W, pbc)

    bond_out = pl.pallas_call(
        _bond_body,
        out_shape=jax.ShapeDtypeStruct((NB, E), jnp.float32),
    )(bf, bond_emb)

    return (hT_out.T, bond_out, graphT.T)


# speedup vs baseline: 2.3441x; 1.0543x over previous
import jax
import jax.numpy as jnp
from jax.experimental import pallas as pl
from jax.experimental.pallas import tpu as pltpu

N = 4096
NB = 8192
E = 32
H = 256
DEPTH = 10
BATCH = 64
NUM_ATOM_TYPES = 200
NUM_BOND_TYPES = 10

_ROWS = 512
_NBLK = N // _ROWS


def _store_h2(h2_ref, xT):
    f32 = jnp.float32
    f8 = jnp.float8_e4m3fn
    hN = jnp.transpose(xT)
    t0 = hN.astype(f8)
    r0 = hN - t0.astype(f32)
    t1 = (r0 * 256.0).astype(f8)
    r1 = r0 - t1.astype(f32) * (1.0 / 256.0)
    t2 = (r1 * 65536.0).astype(f8)
    h2_ref[...] = jnp.concatenate([t0, t1, t2], axis=1)


def _mpn_body(adj_ref, af_ref, bi_ref, atom_embT_ref,
              msgW_ref, mbc_ref, Wih_ref, Whh_ref, bihc_ref, bhhc_ref,
              poolW_ref, pbc_ref,
              hT_out_ref, graphT_ref,
              mask_ref, scale_ref, flag_ref, bufT_ref, h2_ref, agg_ref):
    l = pl.program_id(0)
    b = pl.program_id(1)
    f32 = jnp.float32
    bf16 = jnp.bfloat16
    f8 = jnp.float8_e4m3fn

    @pl.when(jnp.logical_and(l == 0, b == 0))
    def _init():
        af = af_ref[...]
        iota = jax.lax.broadcasted_iota(jnp.int32, (NUM_ATOM_TYPES, N), 0)
        ohT = (iota == af).astype(f32)
        h0T = jnp.dot(atom_embT_ref[...], ohT, preferred_element_type=f32)
        bufT_ref[0] = h0T
        _store_h2(h2_ref, h0T)

    @pl.when(l == 1)
    def _convert():
        a = adj_ref[...]
        m = a != 0
        mask_ref[pl.ds(b * _ROWS, _ROWS), :] = m.astype(f8)
        deg = jnp.sum(m.astype(f32), axis=1, keepdims=True)
        has = deg > 0.0
        safe = jnp.where(has, deg, 1.0)
        sc = jnp.where(has, 1.0 / safe, 0.0)
        scale_ref[:, pl.ds(b * _ROWS, _ROWS)] = jnp.transpose(sc)
        flag_ref[:, pl.ds(b * _ROWS, _ROWS)] = jnp.transpose(
            has.astype(f32))

    @pl.when(l > 0)
    def _layer():
        mblk = mask_ref[pl.ds(b * _ROWS, _ROWS), :]
        acc = jnp.dot(mblk, h2_ref[...], preferred_element_type=f32)
        agg_ref[pl.ds(b * _ROWS, _ROWS), :] = acc

        @pl.when(b == _NBLK - 1)
        def _gates():
            Wih = Wih_ref[0]
            W = msgW_ref[0]
            A_T = jnp.dot(Wih, W[:, :E], preferred_element_type=f32)
            B_T = jnp.dot(Wih, W[:, E:], preferred_element_type=f32)
            top = jnp.concatenate([A_T, B_T], axis=1)
            bot = jnp.concatenate([Whh_ref[0], jnp.zeros((3 * E, E), f32)],
                                  axis=1)
            CT = jnp.concatenate([top, bot], axis=0)
            CThi = CT.astype(bf16)
            CTlo = (CT - CThi.astype(f32)).astype(bf16)
            CT2 = jnp.concatenate([CThi, CTlo], axis=0)
            rowT = jnp.dot(Wih, mbc_ref[0], preferred_element_type=f32)

            aggT = jnp.transpose(agg_ref[...])
            agg1 = (aggT[:E] + aggT[E:2 * E] * (1.0 / 256.0)
                    + aggT[2 * E:] * (1.0 / 65536.0))
            aggsT = agg1 * scale_ref[...]
            hT = bufT_ref[(l - 1) % 2]
            XT = jnp.concatenate([hT, aggsT], axis=0)
            XThi = XT.astype(bf16)
            XTlo = (XT - XThi.astype(f32)).astype(bf16)
            G1 = jnp.dot(CT2, XThi, preferred_element_type=f32)
            G2 = jnp.dot(CThi, XTlo, preferred_element_type=f32)
            GT = G1[:6 * E] + G1[6 * E:] + G2
            giT = flag_ref[...] * (GT[:3 * E] + rowT) + bihc_ref[0]
            ghT = GT[3 * E:] + bhhc_ref[0]
            rT = jax.nn.sigmoid(giT[:E] + ghT[:E])
            zT = jax.nn.sigmoid(giT[E:2 * E] + ghT[E:2 * E])
            nT = jnp.tanh(giT[2 * E:] + rT * ghT[2 * E:])
            hnT = (1.0 - zT) * nT + zT * hT
            bufT_ref[l % 2] = hnT
            _store_h2(h2_ref, hnT)

            @pl.when(l == DEPTH)
            def _emit():
                hT_out_ref[...] = hnT
                bi = bi_ref[...]
                iota = jax.lax.broadcasted_iota(jnp.int32, (N, BATCH), 1)
                PT = (iota == bi).astype(f32)
                countsT = jnp.sum(PT, axis=0, keepdims=True)
                sumsT = jnp.dot(hnT, PT, preferred_element_type=f32)
                inv = jnp.where(countsT > 0.0,
                                1.0 / jnp.where(countsT > 0.0, countsT, 1.0),
                                0.0)
                meansT = sumsT * inv
                graphT_ref[...] = (jnp.dot(poolW_ref[...], meansT,
                                           preferred_element_type=f32)
                                   + pbc_ref[...])


def _bond_body(bf_ref, bond_emb_ref, out_ref):
    bfi = bf_ref[...]
    iota = jax.lax.broadcasted_iota(jnp.int32, (NB, NUM_BOND_TYPES), 1)
    oh = (bfi == iota).astype(jnp.float32)
    out_ref[...] = jnp.dot(oh, bond_emb_ref[...],
                           preferred_element_type=jnp.float32)


def kernel(atom_features, bond_features, adjacency_matrix, batch_indices,
           atom_emb, bond_emb, msg_W, msg_b,
           gru_Wih, gru_Whh, gru_bih, gru_bhh, pool_W, pool_b):
    af = atom_features.reshape(1, N).astype(jnp.int32)
    bf = bond_features.reshape(NB, 1).astype(jnp.int32)
    bi = batch_indices.reshape(N, 1).astype(jnp.int32)
    atom_embT = atom_emb.T
    mbc = msg_b.reshape(DEPTH, H, 1)
    bihc = gru_bih.reshape(DEPTH, 3 * E, 1)
    bhhc = gru_bhh.reshape(DEPTH, 3 * E, 1)
    pbc = pool_b.reshape(H, 1)

    def _wmap(nd):
        def im(l, b):
            lw = jnp.maximum(l - 1, 0)
            return (lw,) + (0,) * (nd - 1)
        return im

    def _adj_map(l, b):
        return (jnp.where(l >= 2, _NBLK - 1, jnp.where(l == 1, b, 0)), 0)

    hT_out, graphT = pl.pallas_call(
        _mpn_body,
        grid=(DEPTH + 1, _NBLK),
        in_specs=[
            pl.BlockSpec((_ROWS, N), _adj_map),
            pl.BlockSpec((1, N), lambda l, b: (0, 0)),
            pl.BlockSpec((N, 1), lambda l, b: (0, 0)),
            pl.BlockSpec((E, NUM_ATOM_TYPES), lambda l, b: (0, 0)),
            pl.BlockSpec((1, H, 2 * E), _wmap(3)),
            pl.BlockSpec((1, H, 1), _wmap(3)),
            pl.BlockSpec((1, 3 * E, H), _wmap(3)),
            pl.BlockSpec((1, 3 * E, E), _wmap(3)),
            pl.BlockSpec((1, 3 * E, 1), _wmap(3)),
            pl.BlockSpec((1, 3 * E, 1), _wmap(3)),
            pl.BlockSpec((H, E), lambda l, b: (0, 0)),
            pl.BlockSpec((H, 1), lambda l, b: (0, 0)),
        ],
        out_specs=[
            pl.BlockSpec((E, N), lambda l, b: (0, 0)),
            pl.BlockSpec((H, BATCH), lambda l, b: (0, 0)),
        ],
        out_shape=[
            jax.ShapeDtypeStruct((E, N), jnp.float32),
            jax.ShapeDtypeStruct((H, BATCH), jnp.float32),
        ],
        scratch_shapes=[
            pltpu.VMEM((N, N), jnp.float8_e4m3fn),
            pltpu.VMEM((1, N), jnp.float32),
            pltpu.VMEM((1, N), jnp.float32),
            pltpu.VMEM((2, E, N), jnp.float32),
            pltpu.VMEM((N, 3 * E), jnp.float8_e4m3fn),
            pltpu.VMEM((N, 3 * E), jnp.float32),
        ],
    )(adjacency_matrix, af, bi, atom_embT, msg_W, mbc, gru_Wih, gru_Whh,
      bihc, bhhc, pool_W, pbc)

    bond_out = pl.pallas_call(
        _bond_body,
        out_shape=jax.ShapeDtypeStruct((NB, E), jnp.float32),
    )(bf, bond_emb)

    return (hT_out.T, bond_out, graphT.T)
